# Initial kernel scaffold; baseline (speedup 1.0000x reference)
import jax
import jax.numpy as jnp
from jax.experimental import pallas as pl


def kernel(x, edge_index, W1, b1, W2, b2, W3, b3, Wf, bf):
    raise NotImplementedError("write your pallas kernel here")


# speedup vs baseline: 8.4335x; 8.4335x over previous
import functools

import jax
import jax.numpy as jnp
from jax import lax
from jax.experimental import pallas as pl
from jax.experimental.pallas import tpu as pltpu
from jax.experimental.pallas import tpu_sc as plsc

_NC = 2
_NS = 16
_C = 64


def _cdiv(a, b):
    return (a + b - 1) // b


def _make_deg_kernel(nacc, NG, G):
    rows_pw = nacc // _NS
    mesh = plsc.VectorSubcoreMesh(core_axis_name="c", subcore_axis_name="s")

    @functools.partial(
        pl.kernel,
        out_type=jax.ShapeDtypeStruct((_NC, nacc), jnp.float32),
        mesh=mesh,
        scratch_types=[
            pltpu.VMEM((NG, G, _C), jnp.int32),
            pltpu.VMEM((_C,), jnp.float32),
            pltpu.VMEM((rows_pw,), jnp.float32),
            pltpu.VMEM_SHARED((nacc,), jnp.float32),
        ],
    )
    def deg_kernel(dst_hbm, ones_hbm, zeros_hbm, out_hbm, dst_v, ones_v, zbuf, acc):
        c = lax.axis_index("c")
        s = lax.axis_index("s")
        w = c * _NS + s
        pltpu.sync_copy(dst_hbm.at[w], dst_v)
        pltpu.sync_copy(ones_hbm, ones_v)
        pltpu.sync_copy(zeros_hbm, zbuf)
        pltpu.sync_copy(zbuf, acc.at[pl.ds(s * rows_pw, rows_pw)])
        plsc.subcore_barrier()

        def group(gi, carry):
            def body(j, carry2):
                pltpu.sync_copy(ones_v, acc.at[dst_v.at[gi, j]], add=True)
                return carry2

            lax.fori_loop(0, G, body, 0)
            return carry

        lax.fori_loop(0, NG, group, 0)
        plsc.subcore_barrier()
        pltpu.sync_copy(acc.at[pl.ds(s * rows_pw, rows_pw)], zbuf)
        pltpu.sync_copy(zbuf, out_hbm.at[c, pl.ds(s * rows_pw, rows_pw)])

    return deg_kernel


def _make_scatter_kernel(nacc, NG, G, d):
    rows_pw = nacc // _NS
    nzc = rows_pw // _C
    mesh = plsc.VectorSubcoreMesh(core_axis_name="c", subcore_axis_name="s")

    @functools.partial(
        pl.kernel,
        out_type=jax.ShapeDtypeStruct((_NC, nacc, d), jnp.float32),
        mesh=mesh,
        compiler_params=pltpu.CompilerParams(use_tc_tiling_on_sc=(d % 128 == 0)),
        scratch_types=[
            pltpu.VMEM((G, _C), jnp.int32),
            pltpu.VMEM((G, _C), jnp.int32),
            pltpu.VMEM((_C, d), jnp.float32),
            pltpu.VMEM((_C, d), jnp.float32),
            pltpu.VMEM_SHARED((nacc, d), jnp.float32),
            pltpu.SemaphoreType.DMA,
            pltpu.SemaphoreType.DMA,
        ],
    )
    def scatter_kernel(src_hbm, dst_hbm, g_hbm, zeros_hbm, out_hbm,
                       src_v, dst_v, buf_a, buf_b, acc, sem_a, sem_b):
        c = lax.axis_index("c")
        s = lax.axis_index("s")
        w = c * _NS + s
        pltpu.sync_copy(zeros_hbm, buf_a)
        for t in range(nzc):
            pltpu.sync_copy(buf_a, acc.at[pl.ds(s * rows_pw + t * _C, _C), :])
        plsc.subcore_barrier()

        def group(gi, carry):
            pltpu.sync_copy(src_hbm.at[w, gi], src_v)
            pltpu.sync_copy(dst_hbm.at[w, gi], dst_v)
            pltpu.async_copy(g_hbm.at[src_v.at[0]], buf_a, sem_a)

            def body(i, carry2):
                ja = 2 * i
                jb = 2 * i + 1
                jn = jnp.minimum(jb + 1, G - 1)
                pltpu.make_async_copy(g_hbm.at[src_v.at[ja]], buf_a, sem_a).wait()
                pltpu.async_copy(g_hbm.at[src_v.at[jb]], buf_b, sem_b)
                pltpu.sync_copy(buf_a, acc.at[dst_v.at[ja]], add=True)
                pltpu.make_async_copy(g_hbm.at[src_v.at[jb]], buf_b, sem_b).wait()
                pltpu.async_copy(g_hbm.at[src_v.at[jn]], buf_a, sem_a)
                pltpu.sync_copy(buf_b, acc.at[dst_v.at[jb]], add=True)
                return carry2

            lax.fori_loop(0, G // 2, body, 0)
            pltpu.make_async_copy(g_hbm.at[src_v.at[G - 1]], buf_a, sem_a).wait()
            return carry

        lax.fori_loop(0, NG, group, 0)
        plsc.subcore_barrier()
        for t in range(nzc):
            r0 = s * rows_pw + t * _C
            pltpu.sync_copy(acc.at[pl.ds(r0, _C), :], buf_a)
            pltpu.sync_copy(buf_a, out_hbm.at[c, pl.ds(r0, _C), :])

    return scatter_kernel


def _tc_first(n):
    def body(x_ref, w_ref, dv_ref, o_ref):
        o_ref[...] = dv_ref[...] * jnp.dot(
            x_ref[...], w_ref[...], preferred_element_type=jnp.float32)

    return body


def _tc_mid(n):
    def body(s_ref, g_ref, dv_ref, b_ref, w_ref, o_ref):
        agg = s_ref[0, pl.ds(0, n), :] + s_ref[1, pl.ds(0, n), :] + g_ref[...]
        h = dv_ref[...] * agg + b_ref[...]
        o_ref[...] = dv_ref[...] * jnp.dot(
            h, w_ref[...], preferred_element_type=jnp.float32)

    return body


def _tc_final(n):
    def body(s_ref, g_ref, dv_ref, b_ref, wf_ref, bf_ref, o_ref):
        agg = s_ref[0, pl.ds(0, n), :] + s_ref[1, pl.ds(0, n), :] + g_ref[...]
        h = dv_ref[...] * agg + b_ref[...]
        logits = jnp.dot(h, wf_ref[...], preferred_element_type=jnp.float32) + bf_ref[...]
        m = jnp.max(logits, axis=1, keepdims=True)
        lse = m + jnp.log(jnp.sum(jnp.exp(logits - m), axis=1, keepdims=True))
        o_ref[...] = logits - lse

    return body


def kernel(x, edge_index, W1, b1, W2, b2, W3, b3, Wf, bf):
    n, d_in = x.shape
    e = edge_index.shape[1]
    hid = W1.shape[1]
    nw = _NC * _NS

    G = 16
    NG = _cdiv(e, nw * _C * G)
    e_pad = nw * NG * G * _C
    nzc = _cdiv(n + 1, _NS * _C)
    rows_pw = nzc * _C
    nacc = rows_pw * _NS

    src = jnp.concatenate(
        [edge_index[0], jnp.zeros((e_pad - e,), jnp.int32)]).reshape(nw, NG, G, _C)
    dst = jnp.concatenate(
        [edge_index[1], jnp.full((e_pad - e,), n, jnp.int32)]).reshape(nw, NG, G, _C)

    ones_c = jnp.ones((_C,), jnp.float32)
    zeros_r = jnp.zeros((rows_pw,), jnp.float32)

    deg_p = _make_deg_kernel(nacc, NG, G)(dst, ones_c, zeros_r)
    deg = deg_p[0, :n] + deg_p[1, :n] + 1.0
    dinv = deg ** -0.5
    dv = dinv[:, None]

    d3 = 32
    W3p = jnp.pad(W3, ((0, 0), (0, d3 - W3.shape[1])))
    b3p = jnp.pad(b3, (0, d3 - b3.shape[0]))
    Wfp = jnp.pad(Wf, ((0, d3 - Wf.shape[0]), (0, 0)))
    out_c = Wf.shape[1]

    scatter_h = _make_scatter_kernel(nacc, NG, G, hid)
    scatter_3 = _make_scatter_kernel(nacc, NG, G, d3)
    zeros_h = jnp.zeros((_C, hid), jnp.float32)
    zeros_3 = jnp.zeros((_C, d3), jnp.float32)

    g1 = pl.pallas_call(
        _tc_first(n),
        out_shape=jax.ShapeDtypeStruct((n, hid), jnp.float32),
    )(x, W1, dv)
    s1 = scatter_h(src, dst, g1, zeros_h)

    g2 = pl.pallas_call(
        _tc_mid(n),
        out_shape=jax.ShapeDtypeStruct((n, hid), jnp.float32),
    )(s1, g1, dv, b1, W2)
    s2 = scatter_h(src, dst, g2, zeros_h)

    g3 = pl.pallas_call(
        _tc_mid(n),
        out_shape=jax.ShapeDtypeStruct((n, d3), jnp.float32),
    )(s2, g2, dv, b2, W3p)
    s3 = scatter_3(src, dst, g3, zeros_3)

    out = pl.pallas_call(
        _tc_final(n),
        out_shape=jax.ShapeDtypeStruct((n, out_c), jnp.float32),
    )(s3, g3, dv, b3p, Wfp, bf)
    return out


# speedup vs baseline: 8.7384x; 1.0361x over previous
import functools

import jax
import jax.numpy as jnp
from jax import lax
from jax.experimental import pallas as pl
from jax.experimental.pallas import tpu as pltpu
from jax.experimental.pallas import tpu_sc as plsc

_NC = 2
_NS = 16
_C = 64


def _cdiv(a, b):
    return (a + b - 1) // b


def _make_deg_kernel(nacc, NG, G):
    rows_pw = nacc // _NS
    mesh = plsc.VectorSubcoreMesh(core_axis_name="c", subcore_axis_name="s")

    @functools.partial(
        pl.kernel,
        out_type=jax.ShapeDtypeStruct((_NC, nacc), jnp.float32),
        mesh=mesh,
        scratch_types=[
            pltpu.VMEM((NG, G, _C), jnp.int32),
            pltpu.VMEM((_C,), jnp.float32),
            pltpu.VMEM((rows_pw,), jnp.float32),
            pltpu.VMEM_SHARED((nacc,), jnp.float32),
        ],
    )
    def deg_kernel(dst_hbm, ones_hbm, zeros_hbm, out_hbm, dst_v, ones_v, zbuf, acc):
        c = lax.axis_index("c")
        s = lax.axis_index("s")
        w = c * _NS + s
        pltpu.sync_copy(dst_hbm.at[w], dst_v)
        pltpu.sync_copy(ones_hbm, ones_v)
        pltpu.sync_copy(zeros_hbm, zbuf)
        pltpu.sync_copy(zbuf, acc.at[pl.ds(s * rows_pw, rows_pw)])
        plsc.subcore_barrier()

        def group(gi, carry):
            def body(j, carry2):
                pltpu.sync_copy(ones_v, acc.at[dst_v.at[gi, j]], add=True)
                return carry2

            lax.fori_loop(0, G, body, 0)
            return carry

        lax.fori_loop(0, NG, group, 0)
        plsc.subcore_barrier()
        pltpu.sync_copy(acc.at[pl.ds(s * rows_pw, rows_pw)], zbuf)
        pltpu.sync_copy(zbuf, out_hbm.at[c, pl.ds(s * rows_pw, rows_pw)])

    return deg_kernel


def _make_scatter_kernel(nacc, NG, G, d):
    rows_pw = nacc // _NS
    nzc = rows_pw // _C
    mesh = plsc.VectorSubcoreMesh(core_axis_name="c", subcore_axis_name="s")

    @functools.partial(
        pl.kernel,
        out_type=jax.ShapeDtypeStruct((_NC, nacc, d), jnp.float32),
        mesh=mesh,
        compiler_params=pltpu.CompilerParams(use_tc_tiling_on_sc=(d % 128 == 0)),
        scratch_types=[
            pltpu.VMEM((G, _C), jnp.int32),
            pltpu.VMEM((G, _C), jnp.int32),
            [pltpu.VMEM((_C, d), jnp.float32) for _ in range(4)],
            pltpu.VMEM_SHARED((nacc, d), jnp.float32),
            [pltpu.SemaphoreType.DMA for _ in range(4)],
            [pltpu.SemaphoreType.DMA for _ in range(4)],
        ],
    )
    def scatter_kernel(src_hbm, dst_hbm, g_hbm, zeros_hbm, out_hbm,
                       src_v, dst_v, bufs, acc, sg, ss):
        c = lax.axis_index("c")
        s = lax.axis_index("s")
        w = c * _NS + s
        pltpu.sync_copy(zeros_hbm, bufs[0])
        for t in range(nzc):
            pltpu.sync_copy(bufs[0], acc.at[pl.ds(s * rows_pw + t * _C, _C), :])
        plsc.subcore_barrier()

        def gather(j, k):
            pltpu.async_copy(g_hbm.at[src_v.at[j]], bufs[k], sg[k])

        def wait_g(j, k):
            pltpu.make_async_copy(g_hbm.at[src_v.at[j]], bufs[k], sg[k]).wait()

        def scat(j, k):
            pltpu.async_copy(bufs[k], acc.at[dst_v.at[j]], ss[k], add=True)

        def wait_s(j, k):
            pltpu.make_async_copy(bufs[k], acc.at[dst_v.at[j]], ss[k]).wait()

        def group(gi, carry):
            pltpu.sync_copy(src_hbm.at[w, gi], src_v)
            pltpu.sync_copy(dst_hbm.at[w, gi], dst_v)
            gather(0, 0)
            gather(1, 1)
            wait_g(0, 0); scat(0, 0); gather(2, 2)
            wait_g(1, 1); scat(1, 1); gather(3, 3)
            wait_g(2, 2); scat(2, 2); wait_s(0, 0); gather(4, 0)
            wait_g(3, 3); scat(3, 3); wait_s(1, 1); gather(5, 1)

            def body(i, carry2):
                j0 = 4 * i
                for k in range(4):
                    j = j0 + k
                    wait_g(j, k)
                    scat(j, k)
                    wait_s(j - 2, (k + 2) % 4)
                    gather(jnp.minimum(j + 2, G - 1), (k + 2) % 4)
                return carry2

            lax.fori_loop(1, G // 4, body, 0)
            wait_s(G - 2, (G - 2) % 4)
            wait_s(G - 1, (G - 1) % 4)
            wait_g(G - 1, G % 4)
            wait_g(G - 1, (G + 1) % 4)
            return carry

        lax.fori_loop(0, NG, group, 0)
        plsc.subcore_barrier()
        for t in range(nzc):
            r0 = s * rows_pw + t * _C
            pltpu.sync_copy(acc.at[pl.ds(r0, _C), :], bufs[0])
            pltpu.sync_copy(bufs[0], out_hbm.at[c, pl.ds(r0, _C), :])

    return scatter_kernel


def _tc_first(n):
    def body(x_ref, w_ref, dv_ref, o_ref):
        o_ref[...] = dv_ref[...] * jnp.dot(
            x_ref[...], w_ref[...], preferred_element_type=jnp.float32)

    return body


def _tc_mid(n):
    def body(s_ref, g_ref, dv_ref, b_ref, w_ref, o_ref):
        agg = s_ref[0, pl.ds(0, n), :] + s_ref[1, pl.ds(0, n), :] + g_ref[...]
        h = dv_ref[...] * agg + b_ref[...]
        o_ref[...] = dv_ref[...] * jnp.dot(
            h, w_ref[...], preferred_element_type=jnp.float32)

    return body


def _tc_final(n):
    def body(s_ref, g_ref, dv_ref, b_ref, wf_ref, bf_ref, o_ref):
        agg = s_ref[0, pl.ds(0, n), :] + s_ref[1, pl.ds(0, n), :] + g_ref[...]
        h = dv_ref[...] * agg + b_ref[...]
        logits = jnp.dot(h, wf_ref[...], preferred_element_type=jnp.float32) + bf_ref[...]
        m = jnp.max(logits, axis=1, keepdims=True)
        lse = m + jnp.log(jnp.sum(jnp.exp(logits - m), axis=1, keepdims=True))
        o_ref[...] = logits - lse

    return body


def kernel(x, edge_index, W1, b1, W2, b2, W3, b3, Wf, bf):
    n, d_in = x.shape
    e = edge_index.shape[1]
    hid = W1.shape[1]
    nw = _NC * _NS

    G = 16
    NG = _cdiv(e, nw * _C * G)
    e_pad = nw * NG * G * _C
    nzc = _cdiv(n + 1, _NS * _C)
    rows_pw = nzc * _C
    nacc = rows_pw * _NS

    src = jnp.concatenate(
        [edge_index[0], jnp.zeros((e_pad - e,), jnp.int32)]).reshape(nw, NG, G, _C)
    dst = jnp.concatenate(
        [edge_index[1], jnp.full((e_pad - e,), n, jnp.int32)]).reshape(nw, NG, G, _C)

    ones_c = jnp.ones((_C,), jnp.float32)
    zeros_r = jnp.zeros((rows_pw,), jnp.float32)

    deg_p = _make_deg_kernel(nacc, NG, G)(dst, ones_c, zeros_r)
    deg = deg_p[0, :n] + deg_p[1, :n] + 1.0
    dinv = deg ** -0.5
    dv = dinv[:, None]

    d3 = 32
    W3p = jnp.pad(W3, ((0, 0), (0, d3 - W3.shape[1])))
    b3p = jnp.pad(b3, (0, d3 - b3.shape[0]))
    Wfp = jnp.pad(Wf, ((0, d3 - Wf.shape[0]), (0, 0)))
    out_c = Wf.shape[1]

    scatter_h = _make_scatter_kernel(nacc, NG, G, hid)
    scatter_3 = _make_scatter_kernel(nacc, NG, G, d3)
    zeros_h = jnp.zeros((_C, hid), jnp.float32)
    zeros_3 = jnp.zeros((_C, d3), jnp.float32)

    g1 = pl.pallas_call(
        _tc_first(n),
        out_shape=jax.ShapeDtypeStruct((n, hid), jnp.float32),
    )(x, W1, dv)
    s1 = scatter_h(src, dst, g1, zeros_h)

    g2 = pl.pallas_call(
        _tc_mid(n),
        out_shape=jax.ShapeDtypeStruct((n, hid), jnp.float32),
    )(s1, g1, dv, b1, W2)
    s2 = scatter_h(src, dst, g2, zeros_h)

    g3 = pl.pallas_call(
        _tc_mid(n),
        out_shape=jax.ShapeDtypeStruct((n, d3), jnp.float32),
    )(s2, g2, dv, b2, W3p)
    s3 = scatter_3(src, dst, g3, zeros_3)

    out = pl.pallas_call(
        _tc_final(n),
        out_shape=jax.ShapeDtypeStruct((n, out_c), jnp.float32),
    )(s3, g3, dv, b3p, Wfp, bf)
    return out


# speedup vs baseline: 8.7415x; 1.0004x over previous
import functools

import jax
import jax.numpy as jnp
from jax import lax
from jax.experimental import pallas as pl
from jax.experimental.pallas import tpu as pltpu
from jax.experimental.pallas import tpu_sc as plsc

_NC = 2
_NS = 16
_C = 64


def _cdiv(a, b):
    return (a + b - 1) // b


def _make_deg_kernel(nacc, NG, G):
    rows_pw = nacc // _NS
    mesh = plsc.VectorSubcoreMesh(core_axis_name="c", subcore_axis_name="s")

    @functools.partial(
        pl.kernel,
        out_type=jax.ShapeDtypeStruct((_NC, nacc), jnp.float32),
        mesh=mesh,
        scratch_types=[
            pltpu.VMEM((NG, G, _C), jnp.int32),
            pltpu.VMEM((_C,), jnp.float32),
            pltpu.VMEM((rows_pw,), jnp.float32),
            pltpu.VMEM_SHARED((nacc,), jnp.float32),
        ],
    )
    def deg_kernel(dst_hbm, ones_hbm, zeros_hbm, out_hbm, dst_v, ones_v, zbuf, acc):
        c = lax.axis_index("c")
        s = lax.axis_index("s")
        w = c * _NS + s
        pltpu.sync_copy(dst_hbm.at[w], dst_v)
        pltpu.sync_copy(ones_hbm, ones_v)
        pltpu.sync_copy(zeros_hbm, zbuf)
        pltpu.sync_copy(zbuf, acc.at[pl.ds(s * rows_pw, rows_pw)])
        plsc.subcore_barrier()

        def group(gi, carry):
            def body(j, carry2):
                pltpu.sync_copy(ones_v, acc.at[dst_v.at[gi, j]], add=True)
                return carry2

            lax.fori_loop(0, G, body, 0)
            return carry

        lax.fori_loop(0, NG, group, 0)
        plsc.subcore_barrier()
        pltpu.sync_copy(acc.at[pl.ds(s * rows_pw, rows_pw)], zbuf)
        pltpu.sync_copy(zbuf, out_hbm.at[c, pl.ds(s * rows_pw, rows_pw)])

    return deg_kernel


def _make_scatter_kernel(nacc, NG, G, d):
    rows_pw = nacc // _NS
    nzc = rows_pw // _C
    mesh = plsc.VectorSubcoreMesh(core_axis_name="c", subcore_axis_name="s")

    @functools.partial(
        pl.kernel,
        out_type=jax.ShapeDtypeStruct((_NC, nacc, d), jnp.float32),
        mesh=mesh,
        compiler_params=pltpu.CompilerParams(use_tc_tiling_on_sc=(d % 128 == 0)),
        scratch_types=[
            pltpu.VMEM((G, _C), jnp.int32),
            pltpu.VMEM((G, _C), jnp.int32),
            [pltpu.VMEM((_C, d), jnp.float32) for _ in range(4)],
            pltpu.VMEM_SHARED((nacc, d), jnp.float32),
            [pltpu.SemaphoreType.DMA for _ in range(4)],
            [pltpu.SemaphoreType.DMA for _ in range(4)],
        ],
    )
    def scatter_kernel(src_hbm, dst_hbm, g_hbm, zeros_hbm, out_hbm,
                       src_v, dst_v, bufs, acc, sg, ss):
        c = lax.axis_index("c")
        s = lax.axis_index("s")
        w = c * _NS + s
        pltpu.sync_copy(zeros_hbm, bufs[0])
        for t in range(nzc):
            pltpu.sync_copy(bufs[0], acc.at[pl.ds(s * rows_pw + t * _C, _C), :])
        plsc.subcore_barrier()

        def gather(j, k):
            pltpu.async_copy(g_hbm.at[src_v.at[j]], bufs[k], sg[k])

        def wait_g(j, k):
            pltpu.make_async_copy(g_hbm.at[src_v.at[j]], bufs[k], sg[k]).wait()

        def scat(j, k):
            pltpu.async_copy(bufs[k], acc.at[dst_v.at[j]], ss[k], add=True)

        def wait_s(j, k):
            pltpu.make_async_copy(bufs[k], acc.at[dst_v.at[j]], ss[k]).wait()

        def group(gi, carry):
            pltpu.sync_copy(src_hbm.at[w, gi], src_v)
            pltpu.sync_copy(dst_hbm.at[w, gi], dst_v)
            gather(0, 0)
            gather(1, 1)
            wait_g(0, 0); scat(0, 0); gather(2, 2)
            wait_g(1, 1); scat(1, 1); gather(3, 3)
            wait_g(2, 2); scat(2, 2); wait_s(0, 0); gather(4, 0)
            wait_g(3, 3); scat(3, 3); wait_s(1, 1); gather(5, 1)

            def body(i, carry2):
                j0 = 4 * i
                for k in range(4):
                    j = j0 + k
                    wait_g(j, k)
                    scat(j, k)
                    wait_s(j - 2, (k + 2) % 4)
                    gather(jnp.minimum(j + 2, G - 1), (k + 2) % 4)
                return carry2

            lax.fori_loop(1, G // 4, body, 0)
            wait_s(G - 2, (G - 2) % 4)
            wait_s(G - 1, (G - 1) % 4)
            wait_g(G - 1, G % 4)
            wait_g(G - 1, (G + 1) % 4)
            return carry

        lax.fori_loop(0, NG, group, 0)
        plsc.subcore_barrier()
        for t in range(nzc):
            r0 = s * rows_pw + t * _C
            pltpu.sync_copy(acc.at[pl.ds(r0, _C), :], bufs[0])
            pltpu.sync_copy(bufs[0], out_hbm.at[c, pl.ds(r0, _C), :])

    return scatter_kernel


def _tc_first(n):
    def body(x_ref, w_ref, dv_ref, o_ref):
        o_ref[...] = dv_ref[...] * jnp.dot(
            x_ref[...], w_ref[...], preferred_element_type=jnp.float32)

    return body


def _tc_mid(n):
    def body(s_ref, g_ref, dv_ref, b_ref, w_ref, o_ref):
        agg = s_ref[0, pl.ds(0, n), :] + s_ref[1, pl.ds(0, n), :] + g_ref[...]
        h = dv_ref[...] * agg + b_ref[...]
        o_ref[...] = dv_ref[...] * jnp.dot(
            h, w_ref[...], preferred_element_type=jnp.float32)

    return body


def _tc_final(n):
    def body(s_ref, g_ref, dv_ref, b_ref, wf_ref, bf_ref, o_ref):
        agg = s_ref[0, pl.ds(0, n), :] + s_ref[1, pl.ds(0, n), :] + g_ref[...]
        h = dv_ref[...] * agg + b_ref[...]
        logits = jnp.dot(h, wf_ref[...], preferred_element_type=jnp.float32) + bf_ref[...]
        m = jnp.max(logits, axis=1, keepdims=True)
        lse = m + jnp.log(jnp.sum(jnp.exp(logits - m), axis=1, keepdims=True))
        o_ref[...] = logits - lse

    return body


def kernel(x, edge_index, W1, b1, W2, b2, W3, b3, Wf, bf):
    n, d_in = x.shape
    e = edge_index.shape[1]
    hid = W1.shape[1]
    nw = _NC * _NS

    G = 16
    NG = _cdiv(e, nw * _C * G)
    e_pad = nw * NG * G * _C
    nzc = _cdiv(n + 1, _NS * _C)
    rows_pw = nzc * _C
    nacc = rows_pw * _NS

    pad_dst = n + jnp.arange(e_pad - e, dtype=jnp.int32) % (nacc - n)
    src = jnp.concatenate(
        [edge_index[0], jnp.zeros((e_pad - e,), jnp.int32)]).reshape(nw, NG, G, _C)
    dst = jnp.concatenate(
        [edge_index[1], pad_dst]).reshape(nw, NG, G, _C)

    ones_c = jnp.ones((_C,), jnp.float32)
    zeros_r = jnp.zeros((rows_pw,), jnp.float32)

    deg_p = _make_deg_kernel(nacc, NG, G)(dst, ones_c, zeros_r)
    deg = deg_p[0, :n] + deg_p[1, :n] + 1.0
    dinv = deg ** -0.5
    dv = dinv[:, None]

    d3 = 32
    W3p = jnp.pad(W3, ((0, 0), (0, d3 - W3.shape[1])))
    b3p = jnp.pad(b3, (0, d3 - b3.shape[0]))
    Wfp = jnp.pad(Wf, ((0, d3 - Wf.shape[0]), (0, 0)))
    out_c = Wf.shape[1]

    scatter_h = _make_scatter_kernel(nacc, NG, G, hid)
    scatter_3 = _make_scatter_kernel(nacc, NG, G, d3)
    zeros_h = jnp.zeros((_C, hid), jnp.float32)
    zeros_3 = jnp.zeros((_C, d3), jnp.float32)

    g1 = pl.pallas_call(
        _tc_first(n),
        out_shape=jax.ShapeDtypeStruct((n, hid), jnp.float32),
    )(x, W1, dv)
    s1 = scatter_h(src, dst, g1, zeros_h)

    g2 = pl.pallas_call(
        _tc_mid(n),
        out_shape=jax.ShapeDtypeStruct((n, hid), jnp.float32),
    )(s1, g1, dv, b1, W2)
    s2 = scatter_h(src, dst, g2, zeros_h)

    g3 = pl.pallas_call(
        _tc_mid(n),
        out_shape=jax.ShapeDtypeStruct((n, d3), jnp.float32),
    )(s2, g2, dv, b2, W3p)
    s3 = scatter_3(src, dst, g3, zeros_3)

    out = pl.pallas_call(
        _tc_final(n),
        out_shape=jax.ShapeDtypeStruct((n, out_c), jnp.float32),
    )(s3, g3, dv, b3p, Wfp, bf)
    return out


# speedup vs baseline: 10.7928x; 1.2347x over previous
import functools

import jax
import jax.numpy as jnp
from jax import lax
from jax.experimental import pallas as pl
from jax.experimental.pallas import tpu as pltpu
from jax.experimental.pallas import tpu_sc as plsc

_NC = 2
_NS = 16
_C = 64


def _cdiv(a, b):
    return (a + b - 1) // b


def _make_deg_kernel(nacc, NPW, G):
    rows_pw = nacc // _NS
    mesh = plsc.VectorSubcoreMesh(core_axis_name="c", subcore_axis_name="s")

    @functools.partial(
        pl.kernel,
        out_type=jax.ShapeDtypeStruct((_NC * nacc,), jnp.float32),
        mesh=mesh,
        scratch_types=[
            pltpu.VMEM((NPW, G, _C), jnp.int32),
            pltpu.VMEM((_C,), jnp.float32),
            pltpu.VMEM((rows_pw,), jnp.float32),
            pltpu.VMEM_SHARED((nacc,), jnp.float32),
        ],
    )
    def deg_kernel(dst_hbm, ones_hbm, zeros_hbm, out_hbm, dst_v, ones_v, zbuf, acc):
        c = lax.axis_index("c")
        s = lax.axis_index("s")
        w = c * _NS + s
        pltpu.sync_copy(dst_hbm.at[w], dst_v)
        pltpu.sync_copy(ones_hbm, ones_v)
        pltpu.sync_copy(zeros_hbm, zbuf)
        pltpu.sync_copy(zbuf, acc.at[pl.ds(s * rows_pw, rows_pw)])
        plsc.subcore_barrier()

        def group(gi, carry):
            def body(j, carry2):
                pltpu.sync_copy(ones_v, acc.at[dst_v.at[gi, j]], add=True)
                return carry2

            lax.fori_loop(0, G, body, 0)
            return carry

        lax.fori_loop(0, NPW, group, 0)
        plsc.subcore_barrier()
        pltpu.sync_copy(acc.at[pl.ds(s * rows_pw, rows_pw)], zbuf)
        pltpu.sync_copy(zbuf, out_hbm.at[pl.ds(c * nacc + s * rows_pw, rows_pw)])

    return deg_kernel


def _make_scatter_kernel(nacc, NG0, NG1, G, d):
    rows_pw = nacc // _NS
    nzfull = rows_pw // _C
    nzrem = rows_pw - nzfull * _C
    mesh = plsc.VectorSubcoreMesh(core_axis_name="c", subcore_axis_name="s")

    @functools.partial(
        pl.kernel,
        out_type=jax.ShapeDtypeStruct((_NC, nacc, d), jnp.float32),
        mesh=mesh,
        compiler_params=pltpu.CompilerParams(use_tc_tiling_on_sc=(d % 128 == 0)),
        scratch_types=[
            pltpu.VMEM((G, _C), jnp.int32),
            pltpu.VMEM((G, _C), jnp.int32),
            [pltpu.VMEM((_C, d), jnp.float32) for _ in range(5)],
            pltpu.VMEM_SHARED((nacc, d), jnp.float32),
            [pltpu.SemaphoreType.DMA for _ in range(5)],
            [pltpu.SemaphoreType.DMA for _ in range(5)],
        ],
    )
    def scatter_kernel(src_hbm, dst_hbm, g_hbm, zeros_hbm, out_hbm,
                       src_v, dst_v, bufs, acc, sg, ss):
        c = lax.axis_index("c")
        s = lax.axis_index("s")
        ngc = jnp.where(c == 0, NG0, NG1)
        gbase = jnp.where(c == 0, s * NG0, _NS * NG0 + s * NG1)
        pltpu.sync_copy(zeros_hbm, bufs[0])
        for t in range(nzfull):
            pltpu.sync_copy(bufs[0], acc.at[pl.ds(s * rows_pw + t * _C, _C), :])
        if nzrem:
            pltpu.sync_copy(bufs[0].at[pl.ds(0, nzrem), :],
                            acc.at[pl.ds(s * rows_pw + nzfull * _C, nzrem), :])
        plsc.subcore_barrier()

        def gather(j, k):
            pltpu.async_copy(g_hbm.at[src_v.at[j]], bufs[k], sg[k])

        def wait_g(j, k):
            pltpu.make_async_copy(g_hbm.at[src_v.at[j]], bufs[k], sg[k]).wait()

        def scat(j, k):
            pltpu.async_copy(bufs[k], acc.at[dst_v.at[j]], ss[k], add=True)

        def wait_s(j, k):
            pltpu.make_async_copy(bufs[k], acc.at[dst_v.at[j]], ss[k]).wait()

        def group(gi, carry):
            gidx = gbase + gi
            pltpu.sync_copy(src_hbm.at[gidx], src_v)
            pltpu.sync_copy(dst_hbm.at[gidx], dst_v)
            gather(0, 0)
            gather(1, 1)
            gather(2, 2)
            for j in range(G):
                wait_g(j, j % 5)
                scat(j, j % 5)
                if j >= 2:
                    wait_s(j - 2, (j - 2) % 5)
                gather(min(j + 3, G - 1), (j + 3) % 5)
            wait_s(G - 2, (G - 2) % 5)
            wait_s(G - 1, (G - 1) % 5)
            for t in range(3):
                wait_g(G - 1, (G + t) % 5)
            return carry

        lax.fori_loop(0, ngc, group, 0)
        plsc.subcore_barrier()
        for t in range(nzfull):
            r0 = s * rows_pw + t * _C
            pltpu.sync_copy(acc.at[pl.ds(r0, _C), :], bufs[0])
            pltpu.sync_copy(bufs[0], out_hbm.at[c, pl.ds(r0, _C), :])
        if nzrem:
            r0 = s * rows_pw + nzfull * _C
            pltpu.sync_copy(acc.at[pl.ds(r0, nzrem), :], bufs[1].at[pl.ds(0, nzrem), :])
            pltpu.sync_copy(bufs[1].at[pl.ds(0, nzrem), :], out_hbm.at[c, pl.ds(r0, nzrem), :])

    return scatter_kernel


def _tc_first(n):
    def body(x_ref, w_ref, dv_ref, o_ref):
        o_ref[...] = dv_ref[...] * jnp.dot(
            x_ref[...], w_ref[...], preferred_element_type=jnp.float32)

    return body


def _tc_mid(n):
    def body(s_ref, g_ref, dv_ref, b_ref, w_ref, o_ref):
        agg = s_ref[0, pl.ds(0, n), :] + s_ref[1, pl.ds(0, n), :] + g_ref[...]
        h = dv_ref[...] * agg + b_ref[...]
        o_ref[...] = dv_ref[...] * jnp.dot(
            h, w_ref[...], preferred_element_type=jnp.float32)

    return body


def _tc_final(n):
    def body(s_ref, g_ref, dv_ref, b_ref, wf_ref, bf_ref, o_ref):
        agg = s_ref[0, pl.ds(0, n), :] + s_ref[1, pl.ds(0, n), :] + g_ref[...]
        h = dv_ref[...] * agg + b_ref[...]
        logits = jnp.dot(h, wf_ref[...], preferred_element_type=jnp.float32) + bf_ref[...]
        m = jnp.max(logits, axis=1, keepdims=True)
        lse = m + jnp.log(jnp.sum(jnp.exp(logits - m), axis=1, keepdims=True))
        o_ref[...] = logits - lse

    return body


def kernel(x, edge_index, W1, b1, W2, b2, W3, b3, Wf, bf):
    n, d_in = x.shape
    e = edge_index.shape[1]
    hid = W1.shape[1]
    nw = _NC * _NS

    G = 32
    npw = _cdiv(e, nw * _C * G)
    ngt = nw * npw
    e_pad = ngt * G * _C
    ng0_h, ng1_h = (2 * npw * 8) // 10, 2 * npw - (2 * npw * 8) // 10
    ng0_3, ng1_3 = (2 * npw * 6) // 10, 2 * npw - (2 * npw * 6) // 10
    rows_pw = 8 * _cdiv(n + 1, _NS * 8)
    nacc = rows_pw * _NS

    pad_dst = n + jnp.arange(e_pad - e, dtype=jnp.int32) % (nacc - n)
    src = jnp.concatenate(
        [edge_index[0], jnp.zeros((e_pad - e,), jnp.int32)]).reshape(ngt, G, _C)
    dst = jnp.concatenate(
        [edge_index[1], pad_dst]).reshape(ngt, G, _C)

    ones_c = jnp.ones((_C,), jnp.float32)
    zeros_r = jnp.zeros((rows_pw,), jnp.float32)

    deg_p = _make_deg_kernel(nacc, npw, G)(
        dst.reshape(nw, npw, G, _C), ones_c, zeros_r)
    deg = deg_p[:n] + deg_p[nacc:nacc + n] + 1.0
    dinv = deg ** -0.5
    dv = dinv[:, None]

    d3 = 32
    W3p = jnp.pad(W3, ((0, 0), (0, d3 - W3.shape[1])))
    b3p = jnp.pad(b3, (0, d3 - b3.shape[0]))
    Wfp = jnp.pad(Wf, ((0, d3 - Wf.shape[0]), (0, 0)))
    out_c = Wf.shape[1]

    scatter_h = _make_scatter_kernel(nacc, ng0_h, ng1_h, G, hid)
    scatter_3 = _make_scatter_kernel(nacc, ng0_3, ng1_3, G, d3)
    zeros_h = jnp.zeros((_C, hid), jnp.float32)
    zeros_3 = jnp.zeros((_C, d3), jnp.float32)

    g1 = pl.pallas_call(
        _tc_first(n),
        out_shape=jax.ShapeDtypeStruct((n, hid), jnp.float32),
    )(x, W1, dv)
    s1 = scatter_h(src, dst, g1, zeros_h)

    g2 = pl.pallas_call(
        _tc_mid(n),
        out_shape=jax.ShapeDtypeStruct((n, hid), jnp.float32),
    )(s1, g1, dv, b1, W2)
    s2 = scatter_h(src, dst, g2, zeros_h)

    g3 = pl.pallas_call(
        _tc_mid(n),
        out_shape=jax.ShapeDtypeStruct((n, d3), jnp.float32),
    )(s2, g2, dv, b2, W3p)
    s3 = scatter_3(src, dst, g3, zeros_3)

    out = pl.pallas_call(
        _tc_final(n),
        out_shape=jax.ShapeDtypeStruct((n, out_c), jnp.float32),
    )(s3, g3, dv, b3p, Wfp, bf)
    return out


# speedup vs baseline: 11.5424x; 1.0694x over previous
import functools

import jax
import jax.numpy as jnp
from jax import lax
from jax.experimental import pallas as pl
from jax.experimental.pallas import tpu as pltpu
from jax.experimental.pallas import tpu_sc as plsc

_NC = 2
_NS = 16
_C = 64


def _cdiv(a, b):
    return (a + b - 1) // b


def _make_deg_kernel(nacc, NPW, G):
    rows_pw = nacc // _NS
    mesh = plsc.VectorSubcoreMesh(core_axis_name="c", subcore_axis_name="s")

    @functools.partial(
        pl.kernel,
        out_type=jax.ShapeDtypeStruct((_NC * nacc,), jnp.float32),
        mesh=mesh,
        scratch_types=[
            pltpu.VMEM((NPW, G, _C), jnp.int32),
            pltpu.VMEM((_C,), jnp.float32),
            pltpu.VMEM((rows_pw,), jnp.float32),
            pltpu.VMEM_SHARED((nacc,), jnp.float32),
        ],
    )
    def deg_kernel(dst_hbm, ones_hbm, zeros_hbm, out_hbm, dst_v, ones_v, zbuf, acc):
        c = lax.axis_index("c")
        s = lax.axis_index("s")
        w = c * _NS + s
        pltpu.sync_copy(dst_hbm.at[w], dst_v)
        pltpu.sync_copy(ones_hbm, ones_v)
        pltpu.sync_copy(zeros_hbm, zbuf)
        pltpu.sync_copy(zbuf, acc.at[pl.ds(s * rows_pw, rows_pw)])
        plsc.subcore_barrier()

        def group(gi, carry):
            def body(j, carry2):
                pltpu.sync_copy(ones_v, acc.at[dst_v.at[gi, j]], add=True)
                return carry2

            lax.fori_loop(0, G, body, 0)
            return carry

        lax.fori_loop(0, NPW, group, 0)
        plsc.subcore_barrier()
        pltpu.sync_copy(acc.at[pl.ds(s * rows_pw, rows_pw)], zbuf)
        pltpu.sync_copy(zbuf, out_hbm.at[pl.ds(c * nacc + s * rows_pw, rows_pw)])

    return deg_kernel


def _make_scatter_kernel(nacc, NG0, NG1, G, d):
    rows_pw = nacc // _NS
    nzfull = rows_pw // _C
    nzrem = rows_pw - nzfull * _C
    mesh = plsc.VectorSubcoreMesh(core_axis_name="c", subcore_axis_name="s")

    @functools.partial(
        pl.kernel,
        out_type=jax.ShapeDtypeStruct((_NC, nacc, d), jnp.float32),
        mesh=mesh,
        compiler_params=pltpu.CompilerParams(use_tc_tiling_on_sc=(d % 128 == 0)),
        scratch_types=[
            pltpu.VMEM((G, _C), jnp.int32),
            pltpu.VMEM((G, _C), jnp.int32),
            [pltpu.VMEM((_C, d), jnp.float32) for _ in range(5)],
            pltpu.VMEM_SHARED((nacc, d), jnp.float32),
            [pltpu.SemaphoreType.DMA for _ in range(5)],
            [pltpu.SemaphoreType.DMA for _ in range(5)],
        ],
    )
    def scatter_kernel(src_hbm, dst_hbm, g_hbm, zeros_hbm, out_hbm,
                       src_v, dst_v, bufs, acc, sg, ss):
        c = lax.axis_index("c")
        s = lax.axis_index("s")
        ngc = jnp.where(c == 0, NG0, NG1)
        gbase = jnp.where(c == 0, s * NG0, _NS * NG0 + s * NG1)
        pltpu.sync_copy(zeros_hbm, bufs[0])
        for t in range(nzfull):
            pltpu.sync_copy(bufs[0], acc.at[pl.ds(s * rows_pw + t * _C, _C), :])
        if nzrem:
            pltpu.sync_copy(bufs[0].at[pl.ds(0, nzrem), :],
                            acc.at[pl.ds(s * rows_pw + nzfull * _C, nzrem), :])
        plsc.subcore_barrier()

        def gather(j, k):
            pltpu.async_copy(g_hbm.at[src_v.at[j]], bufs[k], sg[k])

        def wait_g(j, k):
            pltpu.make_async_copy(g_hbm.at[src_v.at[j]], bufs[k], sg[k]).wait()

        def scat(j, k):
            pltpu.async_copy(bufs[k], acc.at[dst_v.at[j]], ss[k], add=True)

        def wait_s(j, k):
            pltpu.make_async_copy(bufs[k], acc.at[dst_v.at[j]], ss[k]).wait()

        def group(gi, carry):
            gidx = gbase + gi
            pltpu.sync_copy(src_hbm.at[gidx], src_v)
            pltpu.sync_copy(dst_hbm.at[gidx], dst_v)
            gather(0, 0)
            gather(1, 1)
            gather(2, 2)
            for j in range(G):
                wait_g(j, j % 5)
                scat(j, j % 5)
                if j >= 2:
                    wait_s(j - 2, (j - 2) % 5)
                gather(min(j + 3, G - 1), (j + 3) % 5)
            wait_s(G - 2, (G - 2) % 5)
            wait_s(G - 1, (G - 1) % 5)
            for t in range(3):
                wait_g(G - 1, (G + t) % 5)
            return carry

        lax.fori_loop(0, ngc, group, 0)
        plsc.subcore_barrier()
        for t in range(nzfull):
            r0 = s * rows_pw + t * _C
            pltpu.sync_copy(acc.at[pl.ds(r0, _C), :], bufs[0])
            pltpu.sync_copy(bufs[0], out_hbm.at[c, pl.ds(r0, _C), :])
        if nzrem:
            r0 = s * rows_pw + nzfull * _C
            pltpu.sync_copy(acc.at[pl.ds(r0, nzrem), :], bufs[1].at[pl.ds(0, nzrem), :])
            pltpu.sync_copy(bufs[1].at[pl.ds(0, nzrem), :], out_hbm.at[c, pl.ds(r0, nzrem), :])

    return scatter_kernel


def _tc_first(n):
    def body(x_ref, w_ref, dv_ref, o_ref):
        o_ref[...] = dv_ref[...] * jnp.dot(
            x_ref[...], w_ref[...], preferred_element_type=jnp.float32)

    return body


def _tc_mid(n):
    def body(s_ref, g_ref, dv_ref, b_ref, w_ref, o_ref):
        agg = s_ref[0, pl.ds(0, n), :] + s_ref[1, pl.ds(0, n), :] + g_ref[...]
        h = dv_ref[...] * agg + b_ref[...]
        o_ref[...] = dv_ref[...] * jnp.dot(
            h, w_ref[...], preferred_element_type=jnp.float32)

    return body


def _tc_final(n):
    def body(s_ref, g_ref, dv_ref, b_ref, wf_ref, bf_ref, o_ref):
        agg = s_ref[0, pl.ds(0, n), :] + s_ref[1, pl.ds(0, n), :] + g_ref[...]
        h = dv_ref[...] * agg + b_ref[...]
        logits = jnp.dot(h, wf_ref[...], preferred_element_type=jnp.float32) + bf_ref[...]
        m = jnp.max(logits, axis=1, keepdims=True)
        lse = m + jnp.log(jnp.sum(jnp.exp(logits - m), axis=1, keepdims=True))
        o_ref[...] = logits - lse

    return body


def kernel(x, edge_index, W1, b1, W2, b2, W3, b3, Wf, bf):
    n, d_in = x.shape
    e = edge_index.shape[1]
    hid = W1.shape[1]
    nw = _NC * _NS

    G = 32
    npw = _cdiv(e, nw * _C * G)
    ngt = nw * npw
    e_pad = ngt * G * _C
    ng0_h, ng1_h = (2 * npw * 9) // 10, 2 * npw - (2 * npw * 9) // 10
    ng0_3, ng1_3 = (2 * npw * 7) // 10, 2 * npw - (2 * npw * 7) // 10
    rows_pw = 8 * _cdiv(n + 1, _NS * 8)
    nacc = rows_pw * _NS

    pad_dst = n + jnp.arange(e_pad - e, dtype=jnp.int32) % (nacc - n)
    src = jnp.concatenate(
        [edge_index[0], jnp.zeros((e_pad - e,), jnp.int32)]).reshape(ngt, G, _C)
    dst = jnp.concatenate(
        [edge_index[1], pad_dst]).reshape(ngt, G, _C)

    ones_c = jnp.ones((_C,), jnp.float32)
    zeros_r = jnp.zeros((rows_pw,), jnp.float32)

    deg_p = _make_deg_kernel(nacc, npw, G)(
        dst.reshape(nw, npw, G, _C), ones_c, zeros_r)
    deg = deg_p[:n] + deg_p[nacc:nacc + n] + 1.0
    dinv = deg ** -0.5
    dv = dinv[:, None]

    d3 = 32
    W3p = jnp.pad(W3, ((0, 0), (0, d3 - W3.shape[1])))
    b3p = jnp.pad(b3, (0, d3 - b3.shape[0]))
    Wfp = jnp.pad(Wf, ((0, d3 - Wf.shape[0]), (0, 0)))
    out_c = Wf.shape[1]

    scatter_h = _make_scatter_kernel(nacc, ng0_h, ng1_h, G, hid)
    scatter_3 = _make_scatter_kernel(nacc, ng0_3, ng1_3, G, d3)
    zeros_h = jnp.zeros((_C, hid), jnp.float32)
    zeros_3 = jnp.zeros((_C, d3), jnp.float32)

    g1 = pl.pallas_call(
        _tc_first(n),
        out_shape=jax.ShapeDtypeStruct((n, hid), jnp.float32),
    )(x, W1, dv)
    s1 = scatter_h(src, dst, g1, zeros_h)

    g2 = pl.pallas_call(
        _tc_mid(n),
        out_shape=jax.ShapeDtypeStruct((n, hid), jnp.float32),
    )(s1, g1, dv, b1, W2)
    s2 = scatter_h(src, dst, g2, zeros_h)

    g3 = pl.pallas_call(
        _tc_mid(n),
        out_shape=jax.ShapeDtypeStruct((n, d3), jnp.float32),
    )(s2, g2, dv, b2, W3p)
    s3 = scatter_3(src, dst, g3, zeros_3)

    out = pl.pallas_call(
        _tc_final(n),
        out_shape=jax.ShapeDtypeStruct((n, out_c), jnp.float32),
    )(s3, g3, dv, b3p, Wfp, bf)
    return out
